# baseline (speedup 1.0000x reference)
import jax
import jax.numpy as jnp
from jax.experimental import pallas as pl
from jax.experimental.pallas import tpu as pltpu


D_MODEL = 4096
NUM_EXPERTS = 64
TILE_TOK = 1024


def _router_kernel(x_ref, w_ref, b_ref, out_hbm, obuf, osem):
    i = pl.program_id(0)
    n = pl.num_programs(0)
    logits = jax.lax.dot_general(
        x_ref[...], w_ref[...],
        dimension_numbers=(((1,), (1,)), ((), ())),
        preferred_element_type=jnp.float32,
    )
    logits = logits + b_ref[...]
    m = jnp.max(logits, axis=-1, keepdims=True)
    e = jnp.exp(logits - m)
    obuf[pl.ds(i * TILE_TOK, TILE_TOK), :] = e / jnp.sum(e, axis=-1, keepdims=True)

    @pl.when(i == n - 1)
    def _flush():
        copy = pltpu.make_async_copy(obuf, out_hbm, osem)
        copy.start()
        copy.wait()


def kernel(x, W, b):
    n_tok = x.shape[0]
    grid = (n_tok // TILE_TOK,)
    return pl.pallas_call(
        _router_kernel,
        grid=grid,
        in_specs=[
            pl.BlockSpec((TILE_TOK, D_MODEL), lambda i: (i, 0)),
            pl.BlockSpec((NUM_EXPERTS, D_MODEL), lambda i: (0, 0)),
            pl.BlockSpec((NUM_EXPERTS,), lambda i: (0,)),
        ],
        out_specs=pl.BlockSpec(memory_space=pltpu.MemorySpace.HBM),
        out_shape=jax.ShapeDtypeStruct((n_tok, NUM_EXPERTS), jnp.float32),
        scratch_shapes=[
            pltpu.VMEM((32768, NUM_EXPERTS), jnp.float32),
            pltpu.SemaphoreType.DMA,
        ],
        compiler_params=pltpu.CompilerParams(
            dimension_semantics=("arbitrary",),
        ),
    )(x, W, b)
